# Initial kernel scaffold; baseline (speedup 1.0000x reference)
import functools

import jax
import jax.numpy as jnp
from jax import lax
from jax.experimental import pallas as pl
from jax.experimental.pallas import tpu as pltpu
from jax.experimental.pallas import tpu_sc as plsc

NUSR = 25000
NITM = 25000
NN = NUSR + NITM
EMB = 64
UFD = 128
HALF = 32
NP = 50048
NPB = NP // 128
EE = 800000
EPAD = 819200
EB = EPAD // 128
TPB = EB // 16
K = 8
CH = K * 128
NIT = TPB // K
ZR = NP // 16

_F32 = jnp.float32


def _sc_mesh():
    return plsc.VectorSubcoreMesh(
        core_axis_name="c", subcore_axis_name="s", num_cores=2, num_subcores=16
    )


@functools.partial(
    pl.kernel,
    out_type=jax.ShapeDtypeStruct((NP,), _F32),
    mesh=_sc_mesh(),
    scratch_types=[
        pltpu.VMEM((K, 128), jnp.int32),
        pltpu.VMEM((128,), _F32),
        pltpu.VMEM((1024,), _F32),
        pltpu.VMEM_SHARED((NP,), _F32),
    ],
)
def _deg_kernel(colb, deg_out, cbuf, ones, zeros, acc):
    t = lax.axis_index("s")
    for i in range(8):
        ones[pl.ds(i * 16, 16)] = jnp.full((16,), 1.0, _F32)

    def _zb(i, _):
        zeros[pl.ds(i * 16, 16)] = jnp.zeros((16,), _F32)
        return 0

    lax.fori_loop(0, 64, _zb, 0)
    base = t * ZR
    pltpu.sync_copy(zeros.at[:], acc.at[pl.ds(base, 1024)])
    pltpu.sync_copy(zeros.at[:], acc.at[pl.ds(base + 1024, 1024)])
    pltpu.sync_copy(zeros.at[:], acc.at[pl.ds(base + 2048, 1024)])
    pltpu.sync_copy(zeros.at[pl.ds(0, 56)], acc.at[pl.ds(base + 3072, 56)])
    plsc.subcore_barrier()

    def _body(it, _):
        blk0 = t * TPB + it * K
        pltpu.sync_copy(colb.at[pl.ds(blk0, K)], cbuf)
        for j in range(K):
            pltpu.sync_copy(ones, acc.at[cbuf.at[j]], add=True)
        return 0

    lax.fori_loop(0, NIT, _body, 0)
    plsc.subcore_barrier()
    pltpu.sync_copy(acc.at[pl.ds(base, ZR)], deg_out.at[pl.ds(base, ZR)])


@functools.partial(
    pl.kernel,
    out_type=jax.ShapeDtypeStruct((2 * NP, HALF), _F32),
    mesh=_sc_mesh(),
    scratch_types=[
        pltpu.VMEM((K, 128), jnp.int32),
        pltpu.VMEM((K, 128), jnp.int32),
        pltpu.VMEM((K, 128), jnp.int32),
        pltpu.VMEM((CH, HALF), _F32),
        pltpu.VMEM_SHARED((NP, HALF), _F32),
        pltpu.SemaphoreType.DMA,
    ],
)
def _prop_kernel(f2, colb, rowb, g2, cbuf, cbuf2, rbuf, vbuf, acc, sem):
    c = lax.axis_index("c")
    t = lax.axis_index("s")
    coff = c * NP

    def _zb(i, _):
        vbuf[i, pl.ds(0, 16)] = jnp.zeros((16,), _F32)
        vbuf[i, pl.ds(16, 16)] = jnp.zeros((16,), _F32)
        return 0

    lax.fori_loop(0, CH, _zb, 0)
    base = t * ZR
    pltpu.sync_copy(vbuf.at[pl.ds(0, 1024)], acc.at[pl.ds(base, 1024)])
    pltpu.sync_copy(vbuf.at[pl.ds(0, 1024)], acc.at[pl.ds(base + 1024, 1024)])
    pltpu.sync_copy(vbuf.at[pl.ds(0, 1024)], acc.at[pl.ds(base + 2048, 1024)])
    pltpu.sync_copy(vbuf.at[pl.ds(0, 56)], acc.at[pl.ds(base + 3072, 56)])
    plsc.subcore_barrier()

    def _body(it, _):
        blk0 = t * TPB + it * K
        pltpu.sync_copy(colb.at[pl.ds(blk0, K)], cbuf)
        pltpu.sync_copy(rowb.at[pl.ds(blk0, K)], rbuf)
        for j in range(K):
            for i in range(8):
                cbuf2[j, pl.ds(i * 16, 16)] = cbuf[j, pl.ds(i * 16, 16)] + coff
        descs = [
            pltpu.async_copy(
                f2.at[cbuf2.at[j]], vbuf.at[pl.ds(j * 128, 128)], sem
            )
            for j in range(K)
        ]
        for d in descs:
            d.wait()
        for j in range(K):
            pltpu.sync_copy(
                vbuf.at[pl.ds(j * 128, 128)], acc.at[rbuf.at[j]], add=True
            )
        return 0

    lax.fori_loop(0, NIT, _body, 0)
    plsc.subcore_barrier()
    pltpu.sync_copy(acc.at[pl.ds(base, 1024)], g2.at[pl.ds(coff + base, 1024)])
    pltpu.sync_copy(
        acc.at[pl.ds(base + 1024, 1024)], g2.at[pl.ds(coff + base + 1024, 1024)]
    )
    pltpu.sync_copy(
        acc.at[pl.ds(base + 2048, 1024)], g2.at[pl.ds(coff + base + 2048, 1024)]
    )
    pltpu.sync_copy(
        acc.at[pl.ds(base + 3072, 56)], g2.at[pl.ds(coff + base + 3072, 56)]
    )


def _mlp_body(x_ref, e_ref, wft_ref, w1_ref, w2_ref, bf_ref, bc_ref, o_ref):
    p = jnp.maximum(
        jnp.dot(x_ref[...], wft_ref[...], preferred_element_type=_F32)
        + bf_ref[...],
        0.0,
    )
    o = jnp.maximum(
        jnp.dot(e_ref[...], w1_ref[...], preferred_element_type=_F32)
        + jnp.dot(p, w2_ref[...], preferred_element_type=_F32)
        + bc_ref[...],
        0.0,
    )
    o_ref[...] = o


def _mlp(x, emb, wft, w1, w2, bf, bc):
    n = x.shape[0]
    grid = (n + 127) // 128
    return pl.pallas_call(
        _mlp_body,
        grid=(grid,),
        in_specs=[
            pl.BlockSpec((128, UFD), lambda b: (b, 0)),
            pl.BlockSpec((128, EMB), lambda b: (b, 0)),
            pl.BlockSpec((UFD, EMB), lambda b: (0, 0)),
            pl.BlockSpec((EMB, EMB), lambda b: (0, 0)),
            pl.BlockSpec((EMB, EMB), lambda b: (0, 0)),
            pl.BlockSpec((1, EMB), lambda b: (0, 0)),
            pl.BlockSpec((1, EMB), lambda b: (0, 0)),
        ],
        out_specs=pl.BlockSpec((128, EMB), lambda b: (b, 0)),
        out_shape=jax.ShapeDtypeStruct((n, EMB), _F32),
    )(x, emb, wft, w1, w2, bf, bc)


def _prep_body(emb_ref, deg_ref, s_ref, f2_ref):
    b = pl.program_id(1)
    deg = deg_ref[...]
    rows = b * 128 + lax.broadcasted_iota(jnp.int32, (128,), 0)
    valid = (rows < NN) & (deg > 0.0)
    s = jnp.where(valid, lax.rsqrt(jnp.maximum(deg, 1e-30)), 0.0)
    s_ref[...] = s
    f2_ref[...] = s[:, None] * emb_ref[...]


def _prep(emb_p, deg):
    return pl.pallas_call(
        _prep_body,
        grid=(2, NPB),
        in_specs=[
            pl.BlockSpec((128, HALF), lambda h, b: (b, h)),
            pl.BlockSpec((128,), lambda h, b: (b,)),
        ],
        out_specs=[
            pl.BlockSpec((128,), lambda h, b: (b,)),
            pl.BlockSpec((128, HALF), lambda h, b: (h * NPB + b, 0)),
        ],
        out_shape=[
            jax.ShapeDtypeStruct((NP,), _F32),
            jax.ShapeDtypeStruct((2 * NP, HALF), _F32),
        ],
    )(emb_p, deg)


def _accum_body(g_ref, s_ref, acc_ref, oacc_ref, f2_ref):
    s = s_ref[...]
    e = s[:, None] * g_ref[...]
    a = acc_ref[...] + e
    oacc_ref[...] = a
    f2_ref[...] = s[:, None] * e


def _accum_last_body(g_ref, s_ref, acc_ref, fin_ref):
    e = s_ref[...][:, None] * g_ref[...]
    fin_ref[...] = (acc_ref[...] + e) * 0.25


def _accum(g2, s, acc):
    return pl.pallas_call(
        _accum_body,
        grid=(2, NPB),
        in_specs=[
            pl.BlockSpec((128, HALF), lambda h, b: (h * NPB + b, 0)),
            pl.BlockSpec((128,), lambda h, b: (b,)),
            pl.BlockSpec((128, HALF), lambda h, b: (b, h)),
        ],
        out_specs=[
            pl.BlockSpec((128, HALF), lambda h, b: (b, h)),
            pl.BlockSpec((128, HALF), lambda h, b: (h * NPB + b, 0)),
        ],
        out_shape=[
            jax.ShapeDtypeStruct((NP, EMB), _F32),
            jax.ShapeDtypeStruct((2 * NP, HALF), _F32),
        ],
    )(g2, s, acc)


def _accum_last(g2, s, acc):
    return pl.pallas_call(
        _accum_last_body,
        grid=(2, NPB),
        in_specs=[
            pl.BlockSpec((128, HALF), lambda h, b: (h * NPB + b, 0)),
            pl.BlockSpec((128,), lambda h, b: (b,)),
            pl.BlockSpec((128, HALF), lambda h, b: (b, h)),
        ],
        out_specs=pl.BlockSpec((128, HALF), lambda h, b: (b, h)),
        out_shape=jax.ShapeDtypeStruct((NP, EMB), _F32),
    )(g2, s, acc)


@jax.jit
def kernel(edge_index, user_features, item_features, user_emb_table,
           item_emb_table, Wuf, buf, Wuc, buc, Wif, bif, Wic, bic):
    row = edge_index[0]
    col = edge_index[1]
    pad = jnp.full((EPAD - EE,), NN, jnp.int32)
    rowb = jnp.concatenate([row, pad]).reshape(EB, 128)
    colb = jnp.concatenate([col, pad]).reshape(EB, 128)

    u_emb = _mlp(user_features, user_emb_table, Wuf.T,
                 Wuc[:, :EMB].T, Wuc[:, EMB:].T,
                 buf.reshape(1, EMB), buc.reshape(1, EMB))
    i_emb = _mlp(item_features, item_emb_table, Wif.T,
                 Wic[:, :EMB].T, Wic[:, EMB:].T,
                 bif.reshape(1, EMB), bic.reshape(1, EMB))

    deg = _deg_kernel(colb)

    emb_p = jnp.concatenate(
        [u_emb, i_emb, jnp.zeros((NP - NN, EMB), _F32)], axis=0
    )
    s, f2 = _prep(emb_p, deg)

    acc = emb_p
    for layer in range(3):
        g2 = _prop_kernel(f2, colb, rowb)
        if layer < 2:
            acc, f2 = _accum(g2, s, acc)
        else:
            fin = _accum_last(g2, s, acc)
    return fin[:NUSR], fin[NUSR:NN]


# speedup vs baseline: 5.6829x; 5.6829x over previous
import functools

import jax
import jax.numpy as jnp
from jax import lax
from jax.experimental import pallas as pl
from jax.experimental.pallas import tpu as pltpu
from jax.experimental.pallas import tpu_sc as plsc

NUSR = 25000
NITM = 25000
NN = NUSR + NITM
EMB = 64
UFD = 128
HALF = 32
NP = 50176
NPB = NP // 128
EE = 800000
EPAD = 819200
EB = EPAD // 128
TPB = EB // 16
K = 4
CH = K * 128
NIT = TPB // K
ZR = NP // 16
_ZCHUNKS = tuple((i * 512, 512) for i in range(6)) + ((3072, 64),)

_F32 = jnp.float32


def _sc_mesh():
    return plsc.VectorSubcoreMesh(
        core_axis_name="c", subcore_axis_name="s", num_cores=2, num_subcores=16
    )


@functools.partial(
    pl.kernel,
    out_type=jax.ShapeDtypeStruct((NP,), _F32),
    mesh=_sc_mesh(),
    scratch_types=[
        pltpu.VMEM((K, 128), jnp.int32),
        pltpu.VMEM((128,), _F32),
        pltpu.VMEM((1024,), _F32),
        pltpu.VMEM_SHARED((NP,), _F32),
    ],
)
def _deg_kernel(colb, deg_out, cbuf, ones, zeros, acc):
    t = lax.axis_index("s")
    for i in range(8):
        ones[pl.ds(i * 16, 16)] = jnp.full((16,), 1.0, _F32)

    def _zb(i, _):
        zeros[pl.ds(i * 16, 16)] = jnp.zeros((16,), _F32)
        return 0

    lax.fori_loop(0, 64, _zb, 0)
    base = t * ZR
    pltpu.sync_copy(zeros.at[:], acc.at[pl.ds(base, 1024)])
    pltpu.sync_copy(zeros.at[:], acc.at[pl.ds(base + 1024, 1024)])
    pltpu.sync_copy(zeros.at[:], acc.at[pl.ds(base + 2048, 1024)])
    pltpu.sync_copy(zeros.at[pl.ds(0, 64)], acc.at[pl.ds(base + 3072, 64)])
    plsc.subcore_barrier()

    def _body(it, _):
        blk0 = t * TPB + it * K
        pltpu.sync_copy(colb.at[pl.ds(blk0, K)], cbuf)
        for j in range(K):
            pltpu.sync_copy(ones, acc.at[cbuf.at[j]], add=True)
        return 0

    lax.fori_loop(0, NIT, _body, 0)
    plsc.subcore_barrier()
    for off, sz in ((0, 1024), (1024, 1024), (2048, 1024), (3072, 64)):
        pltpu.sync_copy(acc.at[pl.ds(base + off, sz)], zeros.at[pl.ds(0, sz)])
        pltpu.sync_copy(zeros.at[pl.ds(0, sz)], deg_out.at[pl.ds(base + off, sz)])


@functools.partial(
    pl.kernel,
    out_type=jax.ShapeDtypeStruct((2 * NP, HALF), _F32),
    mesh=_sc_mesh(),
    compiler_params=pltpu.CompilerParams(use_tc_tiling_on_sc=False),
    scratch_types=[
        pltpu.VMEM((K, 128), jnp.int32),
        pltpu.VMEM((K, 128), jnp.int32),
        pltpu.VMEM((K, 128), jnp.int32),
        pltpu.VMEM((CH, HALF), _F32),
        pltpu.VMEM_SHARED((NP, HALF), _F32),
        pltpu.SemaphoreType.DMA,
    ],
)
def _prop_kernel(f2, colb, rowb, g2, cbuf, cbuf2, rbuf, vbuf, acc, sem):
    c = lax.axis_index("c")
    t = lax.axis_index("s")
    coff = c * NP

    def _zb(i, _):
        vbuf[i, pl.ds(0, 16)] = jnp.zeros((16,), _F32)
        vbuf[i, pl.ds(16, 16)] = jnp.zeros((16,), _F32)
        return 0

    lax.fori_loop(0, CH, _zb, 0)
    base = t * ZR
    for off, sz in _ZCHUNKS:
        pltpu.sync_copy(vbuf.at[pl.ds(0, sz)], acc.at[pl.ds(base + off, sz)])
    plsc.subcore_barrier()

    def _body(it, _):
        blk0 = t * TPB + it * K
        pltpu.sync_copy(colb.at[pl.ds(blk0, K)], cbuf)
        pltpu.sync_copy(rowb.at[pl.ds(blk0, K)], rbuf)
        for j in range(K):
            for i in range(8):
                cbuf2[j, pl.ds(i * 16, 16)] = cbuf[j, pl.ds(i * 16, 16)] + coff
        descs = [
            pltpu.async_copy(
                f2.at[cbuf2.at[j]], vbuf.at[pl.ds(j * 128, 128)], sem
            )
            for j in range(K)
        ]
        for d in descs:
            d.wait()
        for j in range(K):
            pltpu.sync_copy(
                vbuf.at[pl.ds(j * 128, 128)], acc.at[rbuf.at[j]], add=True
            )
        return 0

    lax.fori_loop(0, NIT, _body, 0)
    plsc.subcore_barrier()
    for off, sz in _ZCHUNKS:
        pltpu.sync_copy(acc.at[pl.ds(base + off, sz)], vbuf.at[pl.ds(0, sz)])
        pltpu.sync_copy(
            vbuf.at[pl.ds(0, sz)], g2.at[pl.ds(coff + base + off, sz)]
        )


def _mlp_body(x_ref, e_ref, wft_ref, w1_ref, w2_ref, bf_ref, bc_ref, o_ref):
    p = jnp.maximum(
        jnp.dot(x_ref[...], wft_ref[...], preferred_element_type=_F32)
        + bf_ref[...],
        0.0,
    )
    o = jnp.maximum(
        jnp.dot(e_ref[...], w1_ref[...], preferred_element_type=_F32)
        + jnp.dot(p, w2_ref[...], preferred_element_type=_F32)
        + bc_ref[...],
        0.0,
    )
    o_ref[...] = o


def _mlp(x, emb, wft, w1, w2, bf, bc):
    n = x.shape[0]
    grid = (n + 127) // 128
    return pl.pallas_call(
        _mlp_body,
        grid=(grid,),
        in_specs=[
            pl.BlockSpec((128, UFD), lambda b: (b, 0)),
            pl.BlockSpec((128, EMB), lambda b: (b, 0)),
            pl.BlockSpec((UFD, EMB), lambda b: (0, 0)),
            pl.BlockSpec((EMB, EMB), lambda b: (0, 0)),
            pl.BlockSpec((EMB, EMB), lambda b: (0, 0)),
            pl.BlockSpec((1, EMB), lambda b: (0, 0)),
            pl.BlockSpec((1, EMB), lambda b: (0, 0)),
        ],
        out_specs=pl.BlockSpec((128, EMB), lambda b: (b, 0)),
        out_shape=jax.ShapeDtypeStruct((n, EMB), _F32),
    )(x, emb, wft, w1, w2, bf, bc)


def _prep_body(emb_ref, deg_ref, s_ref, f2_ref):
    b = pl.program_id(1)
    deg = deg_ref[...]
    rows = b * 128 + lax.broadcasted_iota(jnp.int32, (128,), 0)
    valid = (rows < NN) & (deg > 0.0)
    s = jnp.where(valid, lax.rsqrt(jnp.maximum(deg, 1e-30)), 0.0)
    s_ref[...] = s
    f2_ref[...] = s[:, None] * emb_ref[...]


def _prep(emb2, deg):
    return pl.pallas_call(
        _prep_body,
        grid=(2, NPB),
        in_specs=[
            pl.BlockSpec((128, HALF), lambda h, b: (h * NPB + b, 0)),
            pl.BlockSpec((128,), lambda h, b: (b,)),
        ],
        out_specs=[
            pl.BlockSpec((128,), lambda h, b: (b,)),
            pl.BlockSpec((128, HALF), lambda h, b: (h * NPB + b, 0)),
        ],
        out_shape=[
            jax.ShapeDtypeStruct((NP,), _F32),
            jax.ShapeDtypeStruct((2 * NP, HALF), _F32),
        ],
    )(emb2, deg)


def _accum_body(g_ref, s_ref, acc_ref, oacc_ref, f2_ref):
    s = s_ref[...]
    e = s[:, None] * g_ref[...]
    a = acc_ref[...] + e
    oacc_ref[...] = a
    f2_ref[...] = s[:, None] * e


def _accum_last_body(g_ref, s_ref, acc_ref, fin_ref):
    e = s_ref[...][:, None] * g_ref[...]
    fin_ref[...] = (acc_ref[...] + e) * 0.25


_SPLIT_SPEC = pl.BlockSpec((128, HALF), lambda h, b: (h * NPB + b, 0))
_S_SPEC = pl.BlockSpec((128,), lambda h, b: (b,))


def _accum(g2, s, acc2):
    return pl.pallas_call(
        _accum_body,
        grid=(2, NPB),
        in_specs=[_SPLIT_SPEC, _S_SPEC, _SPLIT_SPEC],
        out_specs=[_SPLIT_SPEC, _SPLIT_SPEC],
        out_shape=[
            jax.ShapeDtypeStruct((2 * NP, HALF), _F32),
            jax.ShapeDtypeStruct((2 * NP, HALF), _F32),
        ],
    )(g2, s, acc2)


def _accum_last(g2, s, acc2):
    return pl.pallas_call(
        _accum_last_body,
        grid=(2, NPB),
        in_specs=[_SPLIT_SPEC, _S_SPEC, _SPLIT_SPEC],
        out_specs=_SPLIT_SPEC,
        out_shape=jax.ShapeDtypeStruct((2 * NP, HALF), _F32),
    )(g2, s, acc2)


@jax.jit
def kernel(edge_index, user_features, item_features, user_emb_table,
           item_emb_table, Wuf, buf, Wuc, buc, Wif, bif, Wic, bic):
    row = edge_index[0]
    col = edge_index[1]
    pad = jnp.full((EPAD - EE,), NN, jnp.int32)
    rowb = jnp.concatenate([row, pad]).reshape(EB, 128)
    colb = jnp.concatenate([col, pad]).reshape(EB, 128)

    u_emb = _mlp(user_features, user_emb_table, Wuf.T,
                 Wuc[:, :EMB].T, Wuc[:, EMB:].T,
                 buf.reshape(1, EMB), buc.reshape(1, EMB))
    i_emb = _mlp(item_features, item_emb_table, Wif.T,
                 Wic[:, :EMB].T, Wic[:, EMB:].T,
                 bif.reshape(1, EMB), bic.reshape(1, EMB))

    deg = _deg_kernel(colb)

    zpad = jnp.zeros((NP - NN, HALF), _F32)
    emb2 = jnp.concatenate(
        [u_emb[:, :HALF], i_emb[:, :HALF], zpad,
         u_emb[:, HALF:], i_emb[:, HALF:], zpad], axis=0
    )
    s, f2 = _prep(emb2, deg)

    acc2 = emb2
    for layer in range(3):
        g2 = _prop_kernel(f2, colb, rowb)
        if layer < 2:
            acc2, f2 = _accum(g2, s, acc2)
        else:
            fin2 = _accum_last(g2, s, acc2)
    users = jnp.concatenate([fin2[:NUSR], fin2[NP:NP + NUSR]], axis=1)
    items = jnp.concatenate(
        [fin2[NUSR:NN], fin2[NP + NUSR:NP + NN]], axis=1
    )
    return users, items


# speedup vs baseline: 9.0178x; 1.5868x over previous
import functools

import jax
import jax.numpy as jnp
from jax import lax
from jax.experimental import pallas as pl
from jax.experimental.pallas import tpu as pltpu
from jax.experimental.pallas import tpu_sc as plsc

NUSR = 25000
NITM = 25000
NN = NUSR + NITM
EMB = 64
UFD = 128
HALF = 32
NP = 50176
NPB = NP // 128
EE = 800000
EPAD = 819200
EB = EPAD // 128
TPB = EB // 16
K = 4
CH = K * 128
NIT = TPB // K
ZR = NP // 16
_ZCHUNKS = tuple((i * 512, 512) for i in range(6)) + ((3072, 64),)

_F32 = jnp.float32


def _sc_mesh():
    return plsc.VectorSubcoreMesh(
        core_axis_name="c", subcore_axis_name="s", num_cores=2, num_subcores=16
    )


@functools.partial(
    pl.kernel,
    out_type=jax.ShapeDtypeStruct((2 * NP,), _F32),
    mesh=_sc_mesh(),
    scratch_types=[
        pltpu.VMEM((K, 128), jnp.int32),
        pltpu.VMEM((128,), _F32),
        pltpu.VMEM((1024,), _F32),
        pltpu.VMEM_SHARED((NP,), _F32),
    ],
)
def _deg_kernel(colb, deg_out, cbuf, ones, zeros, acc):
    c = lax.axis_index("c")
    t = lax.axis_index("s")
    for i in range(8):
        ones[pl.ds(i * 16, 16)] = jnp.full((16,), 1.0, _F32)

    def _zb(i, _):
        zeros[pl.ds(i * 16, 16)] = jnp.zeros((16,), _F32)
        return 0

    lax.fori_loop(0, 64, _zb, 0)
    base = t * ZR
    pltpu.sync_copy(zeros.at[:], acc.at[pl.ds(base, 1024)])
    pltpu.sync_copy(zeros.at[:], acc.at[pl.ds(base + 1024, 1024)])
    pltpu.sync_copy(zeros.at[:], acc.at[pl.ds(base + 2048, 1024)])
    pltpu.sync_copy(zeros.at[pl.ds(0, 64)], acc.at[pl.ds(base + 3072, 64)])
    plsc.subcore_barrier()

    tpb_half = TPB // 2

    def _body(it, _):
        blk0 = c * (EB // 2) + t * tpb_half + it * K
        pltpu.sync_copy(colb.at[pl.ds(blk0, K)], cbuf)
        for j in range(K):
            pltpu.sync_copy(ones, acc.at[cbuf.at[j]], add=True)
        return 0

    lax.fori_loop(0, tpb_half // K, _body, 0)
    plsc.subcore_barrier()
    for off, sz in ((0, 1024), (1024, 1024), (2048, 1024), (3072, 64)):
        pltpu.sync_copy(acc.at[pl.ds(base + off, sz)], zeros.at[pl.ds(0, sz)])
        pltpu.sync_copy(
            zeros.at[pl.ds(0, sz)], deg_out.at[pl.ds(c * NP + base + off, sz)]
        )


@functools.partial(
    pl.kernel,
    out_type=[
        jax.ShapeDtypeStruct((2 * NP, HALF), _F32),
        jax.ShapeDtypeStruct((2 * NP, HALF), _F32),
        jax.ShapeDtypeStruct((2 * NP, HALF), _F32),
        jax.ShapeDtypeStruct((2 * NP, HALF), _F32),
    ],
    mesh=_sc_mesh(),
    compiler_params=pltpu.CompilerParams(use_tc_tiling_on_sc=False),
    scratch_types=[
        pltpu.VMEM((K, 128), jnp.int32),
        pltpu.VMEM((K, 128), jnp.int32),
        pltpu.VMEM((CH, HALF), _F32),
        pltpu.VMEM((ZR + 16,), _F32),
        pltpu.VMEM_SHARED((NP, HALF), _F32),
        pltpu.SemaphoreType.DMA,
    ],
)
def _prop3_kernel(emb2, s, colb, rowb, e1, e2, e3, fb, cbuf, rbuf, vbuf, sv,
                  acc, sem):
    c = lax.axis_index("c")
    t = lax.axis_index("s")
    coff = c * NP
    base = t * ZR

    pltpu.sync_copy(s.at[pl.ds(base, ZR)], sv.at[pl.ds(0, ZR)])

    def _scale_rows(n_rows):
        def _sr(r, off_r):
            sc = jnp.full((16,), sv[pl.ds(off_r + r, 16)][0], _F32)
            vbuf[r, pl.ds(0, 16)] = vbuf[r, pl.ds(0, 16)] * sc
            vbuf[r, pl.ds(16, 16)] = vbuf[r, pl.ds(16, 16)] * sc
            return off_r

        return _sr

    for off, sz in _ZCHUNKS:
        pltpu.sync_copy(emb2.at[pl.ds(coff + base + off, sz)],
                        vbuf.at[pl.ds(0, sz)])
        lax.fori_loop(0, sz, _scale_rows(sz), off)
        pltpu.sync_copy(vbuf.at[pl.ds(0, sz)],
                        fb.at[pl.ds(coff + base + off, sz)])
    plsc.subcore_barrier()

    for layer, e_out in enumerate((e1, e2, e3)):
        def _zb(i, _):
            vbuf[i, pl.ds(0, 16)] = jnp.zeros((16,), _F32)
            vbuf[i, pl.ds(16, 16)] = jnp.zeros((16,), _F32)
            return 0

        lax.fori_loop(0, CH, _zb, 0)
        for off, sz in _ZCHUNKS:
            pltpu.sync_copy(vbuf.at[pl.ds(0, sz)], acc.at[pl.ds(base + off, sz)])
        plsc.subcore_barrier()

        def _body(it, _):
            blk0 = t * TPB + it * K
            pltpu.sync_copy(colb.at[pl.ds(blk0, K)], cbuf)
            pltpu.sync_copy(rowb.at[pl.ds(blk0, K)], rbuf)
            for j in range(K):
                for i in range(8):
                    cbuf[j, pl.ds(i * 16, 16)] = (
                        cbuf[j, pl.ds(i * 16, 16)] + coff
                    )
            descs = [
                pltpu.async_copy(
                    fb.at[cbuf.at[j]], vbuf.at[pl.ds(j * 128, 128)], sem
                )
                for j in range(K)
            ]
            for d in descs:
                d.wait()
            for j in range(K):
                pltpu.sync_copy(
                    vbuf.at[pl.ds(j * 128, 128)], acc.at[rbuf.at[j]], add=True
                )
            return 0

        lax.fori_loop(0, NIT, _body, 0)
        plsc.subcore_barrier()

        for off, sz in _ZCHUNKS:
            pltpu.sync_copy(acc.at[pl.ds(base + off, sz)], vbuf.at[pl.ds(0, sz)])
            lax.fori_loop(0, sz, _scale_rows(sz), off)
            pltpu.sync_copy(vbuf.at[pl.ds(0, sz)],
                            e_out.at[pl.ds(coff + base + off, sz)])
            if layer < 2:
                lax.fori_loop(0, sz, _scale_rows(sz), off)
                pltpu.sync_copy(vbuf.at[pl.ds(0, sz)],
                                fb.at[pl.ds(coff + base + off, sz)])
        if layer < 2:
            plsc.subcore_barrier()


def _mlp_body(x_ref, e_ref, wft_ref, w1_ref, w2_ref, bf_ref, bc_ref, o_ref):
    p = jnp.maximum(
        jnp.dot(x_ref[...], wft_ref[...], preferred_element_type=_F32)
        + bf_ref[...],
        0.0,
    )
    o = jnp.maximum(
        jnp.dot(e_ref[...], w1_ref[...], preferred_element_type=_F32)
        + jnp.dot(p, w2_ref[...], preferred_element_type=_F32)
        + bc_ref[...],
        0.0,
    )
    o_ref[...] = o


def _mlp(x, emb, wft, w1, w2, bf, bc):
    n = x.shape[0]
    grid = (n + 127) // 128
    return pl.pallas_call(
        _mlp_body,
        grid=(grid,),
        in_specs=[
            pl.BlockSpec((128, UFD), lambda b: (b, 0)),
            pl.BlockSpec((128, EMB), lambda b: (b, 0)),
            pl.BlockSpec((UFD, EMB), lambda b: (0, 0)),
            pl.BlockSpec((EMB, EMB), lambda b: (0, 0)),
            pl.BlockSpec((EMB, EMB), lambda b: (0, 0)),
            pl.BlockSpec((1, EMB), lambda b: (0, 0)),
            pl.BlockSpec((1, EMB), lambda b: (0, 0)),
        ],
        out_specs=pl.BlockSpec((128, EMB), lambda b: (b, 0)),
        out_shape=jax.ShapeDtypeStruct((n, EMB), _F32),
    )(x, emb, wft, w1, w2, bf, bc)


def _s_body(d0_ref, d1_ref, s_ref):
    b = pl.program_id(0)
    deg = d0_ref[...] + d1_ref[...]
    rows = b * 512 + lax.broadcasted_iota(jnp.int32, (512,), 0)
    valid = (rows < NN) & (deg > 0.0)
    s_ref[...] = jnp.where(valid, lax.rsqrt(jnp.maximum(deg, 1e-30)), 0.0)


def _s_kernel(degp):
    nb = NP // 512
    return pl.pallas_call(
        _s_body,
        grid=(nb,),
        in_specs=[
            pl.BlockSpec((512,), lambda b: (b,)),
            pl.BlockSpec((512,), lambda b, _nb=nb: (_nb + b,)),
        ],
        out_specs=pl.BlockSpec((512,), lambda b: (b,)),
        out_shape=jax.ShapeDtypeStruct((NP,), _F32),
    )(degp, degp)


def _final_body(a_ref, b_ref, c_ref, d_ref, o_ref):
    o_ref[...] = 0.25 * (a_ref[...] + b_ref[...] + c_ref[...] + d_ref[...])


def _final(emb2, e1, e2, e3):
    rows = 2 * NP * HALF // 128
    view = lambda x: x.reshape(rows, 128)
    spec = pl.BlockSpec((512, 128), lambda b: (b, 0))
    out = pl.pallas_call(
        _final_body,
        grid=(rows // 512,),
        in_specs=[spec, spec, spec, spec],
        out_specs=spec,
        out_shape=jax.ShapeDtypeStruct((rows, 128), _F32),
    )(view(emb2), view(e1), view(e2), view(e3))
    return out.reshape(2 * NP, HALF)


@jax.jit
def kernel(edge_index, user_features, item_features, user_emb_table,
           item_emb_table, Wuf, buf, Wuc, buc, Wif, bif, Wic, bic):
    row = edge_index[0]
    col = edge_index[1]
    pad = jnp.full((EPAD - EE,), NN, jnp.int32)
    rowb = jnp.concatenate([row, pad]).reshape(EB, 128)
    colb = jnp.concatenate([col, pad]).reshape(EB, 128)

    u_emb = _mlp(user_features, user_emb_table, Wuf.T,
                 Wuc[:, :EMB].T, Wuc[:, EMB:].T,
                 buf.reshape(1, EMB), buc.reshape(1, EMB))
    i_emb = _mlp(item_features, item_emb_table, Wif.T,
                 Wic[:, :EMB].T, Wic[:, EMB:].T,
                 bif.reshape(1, EMB), bic.reshape(1, EMB))

    degp = _deg_kernel(colb)
    s = _s_kernel(degp)

    zpad = jnp.zeros((NP - NN, HALF), _F32)
    emb2 = jnp.concatenate(
        [u_emb[:, :HALF], i_emb[:, :HALF], zpad,
         u_emb[:, HALF:], i_emb[:, HALF:], zpad], axis=0
    )

    e1, e2, e3, _ = _prop3_kernel(emb2, s, colb, rowb)

    fin2 = _final(emb2, e1, e2, e3)
    users = jnp.concatenate([fin2[:NUSR], fin2[NP:NP + NUSR]], axis=1)
    items = jnp.concatenate(
        [fin2[NUSR:NN], fin2[NP + NUSR:NP + NN]], axis=1
    )
    return users, items


# speedup vs baseline: 10.2839x; 1.1404x over previous
import functools

import jax
import jax.numpy as jnp
from jax import lax
from jax.experimental import pallas as pl
from jax.experimental.pallas import tpu as pltpu
from jax.experimental.pallas import tpu_sc as plsc

NUSR = 25000
NITM = 25000
NN = NUSR + NITM
EMB = 64
UFD = 128
HALF = 32
NP = 50176
NPB = NP // 128
EE = 800000
EPAD = 819200
EB = EPAD // 128
TPB = EB // 16
K = 4
CH = K * 128
NIT = TPB // K
ZR = NP // 16
_ZCHUNKS = tuple((i * 512, 512) for i in range(6)) + ((3072, 64),)

_F32 = jnp.float32


def _sc_mesh():
    return plsc.VectorSubcoreMesh(
        core_axis_name="c", subcore_axis_name="s", num_cores=2, num_subcores=16
    )


@functools.partial(
    pl.kernel,
    out_type=jax.ShapeDtypeStruct((2 * NP,), _F32),
    mesh=_sc_mesh(),
    scratch_types=[
        pltpu.VMEM((2 * K, 128), jnp.int32),
        pltpu.VMEM((128,), _F32),
        pltpu.VMEM((1024,), _F32),
        pltpu.VMEM_SHARED((NP,), _F32),
        pltpu.SemaphoreType.DMA,
        pltpu.SemaphoreType.DMA,
    ],
)
def _deg_kernel(colb, deg_out, cbuf, ones, zeros, acc, ds0, ds1):
    c = lax.axis_index("c")
    t = lax.axis_index("s")
    for i in range(8):
        ones[pl.ds(i * 16, 16)] = jnp.full((16,), 1.0, _F32)

    def _zb(i, _):
        zeros[pl.ds(i * 16, 16)] = jnp.zeros((16,), _F32)
        return 0

    lax.fori_loop(0, 64, _zb, 0)
    base = t * ZR
    pltpu.sync_copy(zeros.at[:], acc.at[pl.ds(base, 1024)])
    pltpu.sync_copy(zeros.at[:], acc.at[pl.ds(base + 1024, 1024)])
    pltpu.sync_copy(zeros.at[:], acc.at[pl.ds(base + 2048, 1024)])
    pltpu.sync_copy(zeros.at[pl.ds(0, 64)], acc.at[pl.ds(base + 3072, 64)])
    plsc.subcore_barrier()

    tpb_half = TPB // 2
    npair = tpb_half // (2 * K)

    def _load(chunk, slot):
        blk0 = c * (EB // 2) + t * tpb_half + chunk * K
        pltpu.sync_copy(colb.at[pl.ds(blk0, K)], cbuf.at[pl.ds(slot * K, K)])

    def _scat(slot, sem):
        for j in range(K):
            pltpu.async_copy(ones, acc.at[cbuf.at[slot * K + j]], sem,
                             add=True)

    def _wait(sem):
        for _ in range(K):
            pltpu.make_async_copy(
                deg_out.at[pl.ds(0, 128)], zeros.at[pl.ds(0, 128)], sem
            ).wait()

    _load(0, 0)

    def _pair(ip, _):
        a2 = ip * 2
        _scat(0, ds0)

        @pl.when(ip > 0)
        def _():
            _wait(ds1)

        _load(a2 + 1, 1)
        _scat(1, ds1)

        @pl.when(ip < npair - 1)
        def _():
            _wait(ds0)
            _load(a2 + 2, 0)

        return 0

    lax.fori_loop(0, npair, _pair, 0)
    _wait(ds0)
    _wait(ds1)
    plsc.subcore_barrier()
    for off, sz in ((0, 1024), (1024, 1024), (2048, 1024), (3072, 64)):
        pltpu.sync_copy(acc.at[pl.ds(base + off, sz)], zeros.at[pl.ds(0, sz)])
        pltpu.sync_copy(
            zeros.at[pl.ds(0, sz)], deg_out.at[pl.ds(c * NP + base + off, sz)]
        )


@functools.partial(
    pl.kernel,
    out_type=[
        jax.ShapeDtypeStruct((2 * NP, HALF), _F32),
        jax.ShapeDtypeStruct((2 * NP, HALF), _F32),
        jax.ShapeDtypeStruct((2 * NP, HALF), _F32),
        jax.ShapeDtypeStruct((2 * NP, HALF), _F32),
    ],
    mesh=_sc_mesh(),
    compiler_params=pltpu.CompilerParams(use_tc_tiling_on_sc=False),
    scratch_types=[
        pltpu.VMEM((4, 128), jnp.int32),
        pltpu.VMEM((4, 128), jnp.int32),
        pltpu.VMEM((CH, HALF), _F32),
        pltpu.VMEM((ZR + 16,), _F32),
        pltpu.VMEM_SHARED((NP, HALF), _F32),
        pltpu.SemaphoreType.DMA,
        pltpu.SemaphoreType.DMA,
        pltpu.SemaphoreType.DMA,
        pltpu.SemaphoreType.DMA,
    ],
)
def _prop3_kernel(emb2, s, colb, rowb, e1, e2, e3, fb, cbuf, rbuf, vbuf, sv,
                  acc, gs0, gs1, ss0, ss1):
    c = lax.axis_index("c")
    t = lax.axis_index("s")
    coff = c * NP
    base = t * ZR

    pltpu.sync_copy(s.at[pl.ds(base, ZR)], sv.at[pl.ds(0, ZR)])

    def _scale_rows(n_rows):
        def _sr(r, off_r):
            sc = jnp.full((16,), sv[pl.ds(off_r + r, 16)][0], _F32)
            vbuf[r, pl.ds(0, 16)] = vbuf[r, pl.ds(0, 16)] * sc
            vbuf[r, pl.ds(16, 16)] = vbuf[r, pl.ds(16, 16)] * sc
            return off_r

        return _sr

    for off, sz in _ZCHUNKS:
        pltpu.sync_copy(emb2.at[pl.ds(coff + base + off, sz)],
                        vbuf.at[pl.ds(0, sz)])
        lax.fori_loop(0, sz, _scale_rows(sz), off)
        pltpu.sync_copy(vbuf.at[pl.ds(0, sz)],
                        fb.at[pl.ds(coff + base + off, sz)])
    plsc.subcore_barrier()

    for layer, e_out in enumerate((e1, e2, e3)):
        def _zb(i, _):
            vbuf[i, pl.ds(0, 16)] = jnp.zeros((16,), _F32)
            vbuf[i, pl.ds(16, 16)] = jnp.zeros((16,), _F32)
            return 0

        lax.fori_loop(0, CH, _zb, 0)
        for off, sz in _ZCHUNKS:
            pltpu.sync_copy(vbuf.at[pl.ds(0, sz)], acc.at[pl.ds(base + off, sz)])
        plsc.subcore_barrier()

        K2 = 2
        nit2 = TPB // K2
        npair = nit2 // 2

        def _load_idx(chunk, slot):
            blk0 = t * TPB + chunk * K2
            pltpu.sync_copy(colb.at[pl.ds(blk0, K2)],
                            cbuf.at[pl.ds(slot * K2, K2)])
            pltpu.sync_copy(rowb.at[pl.ds(blk0, K2)],
                            rbuf.at[pl.ds(slot * K2, K2)])
            for j in range(K2):
                r = slot * K2 + j
                for i in range(8):
                    cbuf[r, pl.ds(i * 16, 16)] = (
                        cbuf[r, pl.ds(i * 16, 16)] + coff
                    )

        def _gath(slot, sem):
            for j in range(K2):
                pltpu.async_copy(
                    fb.at[cbuf.at[slot * K2 + j]],
                    vbuf.at[pl.ds(slot * 256 + j * 128, 128)], sem
                )

        def _scat(slot, sem):
            for j in range(K2):
                pltpu.async_copy(
                    vbuf.at[pl.ds(slot * 256 + j * 128, 128)],
                    acc.at[rbuf.at[slot * K2 + j]], sem, add=True
                )

        def _wait2(sem):
            for _ in range(K2):
                pltpu.make_async_copy(
                    fb.at[pl.ds(0, 128)], vbuf.at[pl.ds(0, 128)], sem
                ).wait()

        _load_idx(0, 0)
        _gath(0, gs0)

        def _pair(ip, _):
            a2 = ip * 2

            @pl.when(ip > 0)
            def _():
                _wait2(ss1)

            _load_idx(a2 + 1, 1)
            _gath(1, gs1)
            _wait2(gs0)
            _scat(0, ss0)

            @pl.when(ip < npair - 1)
            def _():
                _wait2(ss0)
                _load_idx(a2 + 2, 0)
                _gath(0, gs0)

            _wait2(gs1)
            _scat(1, ss1)
            return 0

        lax.fori_loop(0, npair, _pair, 0)
        _wait2(ss0)
        _wait2(ss1)
        plsc.subcore_barrier()

        for off, sz in _ZCHUNKS:
            pltpu.sync_copy(acc.at[pl.ds(base + off, sz)], vbuf.at[pl.ds(0, sz)])
            lax.fori_loop(0, sz, _scale_rows(sz), off)
            pltpu.sync_copy(vbuf.at[pl.ds(0, sz)],
                            e_out.at[pl.ds(coff + base + off, sz)])
            if layer < 2:
                lax.fori_loop(0, sz, _scale_rows(sz), off)
                pltpu.sync_copy(vbuf.at[pl.ds(0, sz)],
                                fb.at[pl.ds(coff + base + off, sz)])
        if layer < 2:
            plsc.subcore_barrier()


def _mlp_body(x_ref, e_ref, wft_ref, w1_ref, w2_ref, bf_ref, bc_ref, o_ref):
    p = jnp.maximum(
        jnp.dot(x_ref[...], wft_ref[...], preferred_element_type=_F32)
        + bf_ref[...],
        0.0,
    )
    o = jnp.maximum(
        jnp.dot(e_ref[...], w1_ref[...], preferred_element_type=_F32)
        + jnp.dot(p, w2_ref[...], preferred_element_type=_F32)
        + bc_ref[...],
        0.0,
    )
    o_ref[...] = o


def _mlp(x, emb, wft, w1, w2, bf, bc):
    n = x.shape[0]
    grid = (n + 127) // 128
    return pl.pallas_call(
        _mlp_body,
        grid=(grid,),
        in_specs=[
            pl.BlockSpec((128, UFD), lambda b: (b, 0)),
            pl.BlockSpec((128, EMB), lambda b: (b, 0)),
            pl.BlockSpec((UFD, EMB), lambda b: (0, 0)),
            pl.BlockSpec((EMB, EMB), lambda b: (0, 0)),
            pl.BlockSpec((EMB, EMB), lambda b: (0, 0)),
            pl.BlockSpec((1, EMB), lambda b: (0, 0)),
            pl.BlockSpec((1, EMB), lambda b: (0, 0)),
        ],
        out_specs=pl.BlockSpec((128, EMB), lambda b: (b, 0)),
        out_shape=jax.ShapeDtypeStruct((n, EMB), _F32),
    )(x, emb, wft, w1, w2, bf, bc)


def _s_body(d0_ref, d1_ref, s_ref):
    b = pl.program_id(0)
    deg = d0_ref[...] + d1_ref[...]
    rows = b * 512 + lax.broadcasted_iota(jnp.int32, (512,), 0)
    valid = (rows < NN) & (deg > 0.0)
    s_ref[...] = jnp.where(valid, lax.rsqrt(jnp.maximum(deg, 1e-30)), 0.0)


def _s_kernel(degp):
    nb = NP // 512
    return pl.pallas_call(
        _s_body,
        grid=(nb,),
        in_specs=[
            pl.BlockSpec((512,), lambda b: (b,)),
            pl.BlockSpec((512,), lambda b, _nb=nb: (_nb + b,)),
        ],
        out_specs=pl.BlockSpec((512,), lambda b: (b,)),
        out_shape=jax.ShapeDtypeStruct((NP,), _F32),
    )(degp, degp)


def _final_body(a_ref, b_ref, c_ref, d_ref, o_ref):
    o_ref[...] = 0.25 * (a_ref[...] + b_ref[...] + c_ref[...] + d_ref[...])


def _final(emb2, e1, e2, e3):
    rows = 2 * NP * HALF // 128
    view = lambda x: x.reshape(rows, 128)
    spec = pl.BlockSpec((512, 128), lambda b: (b, 0))
    out = pl.pallas_call(
        _final_body,
        grid=(rows // 512,),
        in_specs=[spec, spec, spec, spec],
        out_specs=spec,
        out_shape=jax.ShapeDtypeStruct((rows, 128), _F32),
    )(view(emb2), view(e1), view(e2), view(e3))
    return out.reshape(2 * NP, HALF)


@jax.jit
def kernel(edge_index, user_features, item_features, user_emb_table,
           item_emb_table, Wuf, buf, Wuc, buc, Wif, bif, Wic, bic):
    row = edge_index[0]
    col = edge_index[1]
    pad = jnp.full((EPAD - EE,), NN, jnp.int32)
    rowb = jnp.concatenate([row, pad]).reshape(EB, 128)
    colb = jnp.concatenate([col, pad]).reshape(EB, 128)

    u_emb = _mlp(user_features, user_emb_table, Wuf.T,
                 Wuc[:, :EMB].T, Wuc[:, EMB:].T,
                 buf.reshape(1, EMB), buc.reshape(1, EMB))
    i_emb = _mlp(item_features, item_emb_table, Wif.T,
                 Wic[:, :EMB].T, Wic[:, EMB:].T,
                 bif.reshape(1, EMB), bic.reshape(1, EMB))

    degp = _deg_kernel(colb)
    s = _s_kernel(degp)

    zpad = jnp.zeros((NP - NN, HALF), _F32)
    emb2 = jnp.concatenate(
        [u_emb[:, :HALF], i_emb[:, :HALF], zpad,
         u_emb[:, HALF:], i_emb[:, HALF:], zpad], axis=0
    )

    e1, e2, e3, _ = _prop3_kernel(emb2, s, colb, rowb)

    fin2 = _final(emb2, e1, e2, e3)
    users = jnp.concatenate([fin2[:NUSR], fin2[NP:NP + NUSR]], axis=1)
    items = jnp.concatenate(
        [fin2[NUSR:NN], fin2[NP + NUSR:NP + NN]], axis=1
    )
    return users, items


# speedup vs baseline: 10.4928x; 1.0203x over previous
import functools

import jax
import jax.numpy as jnp
from jax import lax
from jax.experimental import pallas as pl
from jax.experimental.pallas import tpu as pltpu
from jax.experimental.pallas import tpu_sc as plsc

NUSR = 25000
NITM = 25000
NN = NUSR + NITM
EMB = 64
UFD = 128
HALF = 32
NP = 50176
NPB = NP // 128
EE = 800000
EPAD = 819200
EB = EPAD // 128
TPB = EB // 16
K = 4
ZR = NP // 16
CHK = 320
ETILE = EPAD // 16
NCH = ETILE // CHK
NPAIR = NCH // 2
_ZCHUNKS = tuple((i * CHK, CHK) for i in range(9)) + ((2880, 256),)

_F32 = jnp.float32


def _sc_mesh():
    return plsc.VectorSubcoreMesh(
        core_axis_name="c", subcore_axis_name="s", num_cores=2, num_subcores=16
    )


@functools.partial(
    pl.kernel,
    out_type=jax.ShapeDtypeStruct((2 * NP,), _F32),
    mesh=_sc_mesh(),
    scratch_types=[
        pltpu.VMEM((2 * K, 128), jnp.int32),
        pltpu.VMEM((128,), _F32),
        pltpu.VMEM((1024,), _F32),
        pltpu.VMEM_SHARED((NP,), _F32),
        pltpu.SemaphoreType.DMA,
        pltpu.SemaphoreType.DMA,
    ],
)
def _deg_kernel(colb, deg_out, cbuf, ones, zeros, acc, ds0, ds1):
    c = lax.axis_index("c")
    t = lax.axis_index("s")
    for i in range(8):
        ones[pl.ds(i * 16, 16)] = jnp.full((16,), 1.0, _F32)

    def _zb(i, _):
        zeros[pl.ds(i * 16, 16)] = jnp.zeros((16,), _F32)
        return 0

    lax.fori_loop(0, 64, _zb, 0)
    base = t * ZR
    pltpu.sync_copy(zeros.at[:], acc.at[pl.ds(base, 1024)])
    pltpu.sync_copy(zeros.at[:], acc.at[pl.ds(base + 1024, 1024)])
    pltpu.sync_copy(zeros.at[:], acc.at[pl.ds(base + 2048, 1024)])
    pltpu.sync_copy(zeros.at[pl.ds(0, 64)], acc.at[pl.ds(base + 3072, 64)])
    plsc.subcore_barrier()

    tpb_half = TPB // 2
    npair = tpb_half // (2 * K)

    def _load(chunk, slot):
        blk0 = c * (EB // 2) + t * tpb_half + chunk * K
        pltpu.sync_copy(colb.at[pl.ds(blk0, K)], cbuf.at[pl.ds(slot * K, K)])

    def _scat(slot, sem):
        for j in range(K):
            pltpu.async_copy(ones, acc.at[cbuf.at[slot * K + j]], sem,
                             add=True)

    def _wait(sem):
        for _ in range(K):
            pltpu.make_async_copy(
                deg_out.at[pl.ds(0, 128)], zeros.at[pl.ds(0, 128)], sem
            ).wait()

    _load(0, 0)

    def _pair(ip, _):
        a2 = ip * 2
        _scat(0, ds0)

        @pl.when(ip > 0)
        def _():
            _wait(ds1)

        _load(a2 + 1, 1)
        _scat(1, ds1)

        @pl.when(ip < npair - 1)
        def _():
            _wait(ds0)
            _load(a2 + 2, 0)

        return 0

    lax.fori_loop(0, npair, _pair, 0)
    _wait(ds0)
    _wait(ds1)
    plsc.subcore_barrier()
    for off, sz in ((0, 1024), (1024, 1024), (2048, 1024), (3072, 64)):
        pltpu.sync_copy(acc.at[pl.ds(base + off, sz)], zeros.at[pl.ds(0, sz)])
        pltpu.sync_copy(
            zeros.at[pl.ds(0, sz)], deg_out.at[pl.ds(c * NP + base + off, sz)]
        )


@functools.partial(
    pl.kernel,
    out_type=[
        jax.ShapeDtypeStruct((2 * NP, HALF), _F32),
        jax.ShapeDtypeStruct((2 * NP, HALF), _F32),
        jax.ShapeDtypeStruct((2 * NP, HALF), _F32),
        jax.ShapeDtypeStruct((2 * NP, HALF), _F32),
    ],
    mesh=_sc_mesh(),
    compiler_params=pltpu.CompilerParams(use_tc_tiling_on_sc=False),
    scratch_types=[
        pltpu.VMEM((CHK,), jnp.int32),
        pltpu.VMEM((CHK,), jnp.int32),
        pltpu.VMEM((CHK,), jnp.int32),
        pltpu.VMEM((CHK,), jnp.int32),
        pltpu.VMEM((CHK, HALF), _F32),
        pltpu.VMEM((CHK, HALF), _F32),
        pltpu.VMEM((ZR + 16,), _F32),
        pltpu.VMEM_SHARED((NP, HALF), _F32),
        pltpu.SemaphoreType.DMA,
        pltpu.SemaphoreType.DMA,
        pltpu.SemaphoreType.DMA,
        pltpu.SemaphoreType.DMA,
    ],
)
def _prop3_kernel(emb2, s, colf, rowf, e1, e2, e3, fb, cbufA, cbufB, rbufA,
                  rbufB, vbufA, vbufB, sv, acc, gs0, gs1, ss0, ss1):
    c = lax.axis_index("c")
    t = lax.axis_index("s")
    coff = c * NP
    base = t * ZR

    pltpu.sync_copy(s.at[pl.ds(base, ZR)], sv.at[pl.ds(0, ZR)])

    def _scale_rows(n_rows):
        def _sr(r, off_r):
            sc = jnp.full((16,), sv[pl.ds(off_r + r, 16)][0], _F32)
            vbufA[r, pl.ds(0, 16)] = vbufA[r, pl.ds(0, 16)] * sc
            vbufA[r, pl.ds(16, 16)] = vbufA[r, pl.ds(16, 16)] * sc
            return off_r

        return _sr

    for off, sz in _ZCHUNKS:
        pltpu.sync_copy(emb2.at[pl.ds(coff + base + off, sz)],
                        vbufA.at[pl.ds(0, sz)])
        lax.fori_loop(0, sz, _scale_rows(sz), off)
        pltpu.sync_copy(vbufA.at[pl.ds(0, sz)],
                        fb.at[pl.ds(coff + base + off, sz)])
    plsc.subcore_barrier()

    ebase = t * ETILE

    def _load(chunk, cb, rb):
        eoff = ebase + chunk * CHK
        pltpu.sync_copy(colf.at[pl.ds(c * EPAD + eoff, CHK)], cb)
        pltpu.sync_copy(rowf.at[pl.ds(eoff, CHK)], rb)

    def _wait1(sem):
        pltpu.make_async_copy(fb.at[pl.ds(0, CHK)], vbufA, sem).wait()

    for layer, e_out in enumerate((e1, e2, e3)):
        def _zb(i, _):
            vbufA[i, pl.ds(0, 16)] = jnp.zeros((16,), _F32)
            vbufA[i, pl.ds(16, 16)] = jnp.zeros((16,), _F32)
            return 0

        lax.fori_loop(0, CHK, _zb, 0)
        for off, sz in _ZCHUNKS:
            pltpu.sync_copy(vbufA.at[pl.ds(0, sz)],
                            acc.at[pl.ds(base + off, sz)])
        plsc.subcore_barrier()

        _load(0, cbufA, rbufA)
        pltpu.async_copy(fb.at[cbufA], vbufA, gs0)

        def _pair(ip, _):
            a2 = ip * 2

            @pl.when(ip > 0)
            def _():
                _wait1(ss1)

            _load(a2 + 1, cbufB, rbufB)
            pltpu.async_copy(fb.at[cbufB], vbufB, gs1)
            _wait1(gs0)
            pltpu.async_copy(vbufA, acc.at[rbufA], ss0, add=True)

            @pl.when(ip < NPAIR - 1)
            def _():
                _wait1(ss0)
                _load(a2 + 2, cbufA, rbufA)
                pltpu.async_copy(fb.at[cbufA], vbufA, gs0)

            _wait1(gs1)
            pltpu.async_copy(vbufB, acc.at[rbufB], ss1, add=True)
            return 0

        lax.fori_loop(0, NPAIR, _pair, 0)
        _wait1(ss0)
        _wait1(ss1)
        plsc.subcore_barrier()

        for off, sz in _ZCHUNKS:
            pltpu.sync_copy(acc.at[pl.ds(base + off, sz)],
                            vbufA.at[pl.ds(0, sz)])
            lax.fori_loop(0, sz, _scale_rows(sz), off)
            pltpu.sync_copy(vbufA.at[pl.ds(0, sz)],
                            e_out.at[pl.ds(coff + base + off, sz)])
            if layer < 2:
                lax.fori_loop(0, sz, _scale_rows(sz), off)
                pltpu.sync_copy(vbufA.at[pl.ds(0, sz)],
                                fb.at[pl.ds(coff + base + off, sz)])
        if layer < 2:
            plsc.subcore_barrier()


def _mlp_body(x_ref, e_ref, wft_ref, w1_ref, w2_ref, bf_ref, bc_ref, o_ref):
    p = jnp.maximum(
        jnp.dot(x_ref[...], wft_ref[...], preferred_element_type=_F32)
        + bf_ref[...],
        0.0,
    )
    o = jnp.maximum(
        jnp.dot(e_ref[...], w1_ref[...], preferred_element_type=_F32)
        + jnp.dot(p, w2_ref[...], preferred_element_type=_F32)
        + bc_ref[...],
        0.0,
    )
    o_ref[...] = o


def _mlp(x, emb, wft, w1, w2, bf, bc):
    n = x.shape[0]
    grid = (n + 127) // 128
    return pl.pallas_call(
        _mlp_body,
        grid=(grid,),
        in_specs=[
            pl.BlockSpec((128, UFD), lambda b: (b, 0)),
            pl.BlockSpec((128, EMB), lambda b: (b, 0)),
            pl.BlockSpec((UFD, EMB), lambda b: (0, 0)),
            pl.BlockSpec((EMB, EMB), lambda b: (0, 0)),
            pl.BlockSpec((EMB, EMB), lambda b: (0, 0)),
            pl.BlockSpec((1, EMB), lambda b: (0, 0)),
            pl.BlockSpec((1, EMB), lambda b: (0, 0)),
        ],
        out_specs=pl.BlockSpec((128, EMB), lambda b: (b, 0)),
        out_shape=jax.ShapeDtypeStruct((n, EMB), _F32),
    )(x, emb, wft, w1, w2, bf, bc)


def _s_body(d0_ref, d1_ref, s_ref):
    b = pl.program_id(0)
    deg = d0_ref[...] + d1_ref[...]
    rows = b * 512 + lax.broadcasted_iota(jnp.int32, (512,), 0)
    valid = (rows < NN) & (deg > 0.0)
    s_ref[...] = jnp.where(valid, lax.rsqrt(jnp.maximum(deg, 1e-30)), 0.0)


def _s_kernel(degp):
    nb = NP // 512
    return pl.pallas_call(
        _s_body,
        grid=(nb,),
        in_specs=[
            pl.BlockSpec((512,), lambda b: (b,)),
            pl.BlockSpec((512,), lambda b, _nb=nb: (_nb + b,)),
        ],
        out_specs=pl.BlockSpec((512,), lambda b: (b,)),
        out_shape=jax.ShapeDtypeStruct((NP,), _F32),
    )(degp, degp)


def _final_body(a_ref, b_ref, c_ref, d_ref, o_ref):
    o_ref[...] = 0.25 * (a_ref[...] + b_ref[...] + c_ref[...] + d_ref[...])


def _final(emb2, e1, e2, e3):
    rows = 2 * NP * HALF // 128
    view = lambda x: x.reshape(rows, 128)
    spec = pl.BlockSpec((512, 128), lambda b: (b, 0))
    out = pl.pallas_call(
        _final_body,
        grid=(rows // 512,),
        in_specs=[spec, spec, spec, spec],
        out_specs=spec,
        out_shape=jax.ShapeDtypeStruct((rows, 128), _F32),
    )(view(emb2), view(e1), view(e2), view(e3))
    return out.reshape(2 * NP, HALF)


@jax.jit
def kernel(edge_index, user_features, item_features, user_emb_table,
           item_emb_table, Wuf, buf, Wuc, buc, Wif, bif, Wic, bic):
    row = edge_index[0]
    col = edge_index[1]
    pad = jnp.full((EPAD - EE,), NN, jnp.int32)
    colb = jnp.concatenate([col, pad]).reshape(EB, 128)

    u_emb = _mlp(user_features, user_emb_table, Wuf.T,
                 Wuc[:, :EMB].T, Wuc[:, EMB:].T,
                 buf.reshape(1, EMB), buc.reshape(1, EMB))
    i_emb = _mlp(item_features, item_emb_table, Wif.T,
                 Wic[:, :EMB].T, Wic[:, EMB:].T,
                 bif.reshape(1, EMB), bic.reshape(1, EMB))

    degp = _deg_kernel(colb)
    s = _s_kernel(degp)

    zpad = jnp.zeros((NP - NN, HALF), _F32)
    emb2 = jnp.concatenate(
        [u_emb[:, :HALF], i_emb[:, :HALF], zpad,
         u_emb[:, HALF:], i_emb[:, HALF:], zpad], axis=0
    )

    colp = jnp.concatenate([col, pad])
    rowp = jnp.concatenate([row, pad])
    colf = jnp.concatenate([colp, colp + NP])
    e1, e2, e3, _ = _prop3_kernel(emb2, s, colf, rowp)

    fin2 = _final(emb2, e1, e2, e3)
    users = jnp.concatenate([fin2[:NUSR], fin2[NP:NP + NUSR]], axis=1)
    items = jnp.concatenate(
        [fin2[NUSR:NN], fin2[NP + NUSR:NP + NN]], axis=1
    )
    return users, items


# speedup vs baseline: 11.3178x; 1.0786x over previous
import functools

import jax
import jax.numpy as jnp
from jax import lax
from jax.experimental import pallas as pl
from jax.experimental.pallas import tpu as pltpu
from jax.experimental.pallas import tpu_sc as plsc

NUSR = 25000
NITM = 25000
NN = NUSR + NITM
EMB = 64
UFD = 128
HALF = 32
NP = 50176
NPB = NP // 128
EE = 800000
EPAD = 819200
EB = EPAD // 128
TPB = EB // 16
K = 4
ZR = NP // 16
CHK = 320
ETILE = EPAD // 16
NCH = ETILE // CHK
NPAIR = NCH // 2
_ZCHUNKS = tuple((i * CHK, CHK) for i in range(9)) + ((2880, 256),)

_F32 = jnp.float32


def _sc_mesh():
    return plsc.VectorSubcoreMesh(
        core_axis_name="c", subcore_axis_name="s", num_cores=2, num_subcores=16
    )


@functools.partial(
    pl.kernel,
    out_type=jax.ShapeDtypeStruct((2 * NP,), _F32),
    mesh=_sc_mesh(),
    scratch_types=[
        pltpu.VMEM((2 * K, 128), jnp.int32),
        pltpu.VMEM((128,), _F32),
        pltpu.VMEM((1024,), _F32),
        pltpu.VMEM_SHARED((NP,), _F32),
        pltpu.SemaphoreType.DMA,
        pltpu.SemaphoreType.DMA,
    ],
)
def _deg_kernel(colb, deg_out, cbuf, ones, zeros, acc, ds0, ds1):
    c = lax.axis_index("c")
    t = lax.axis_index("s")
    for i in range(8):
        ones[pl.ds(i * 16, 16)] = jnp.full((16,), 1.0, _F32)

    def _zb(i, _):
        zeros[pl.ds(i * 16, 16)] = jnp.zeros((16,), _F32)
        return 0

    lax.fori_loop(0, 64, _zb, 0)
    base = t * ZR
    pltpu.sync_copy(zeros.at[:], acc.at[pl.ds(base, 1024)])
    pltpu.sync_copy(zeros.at[:], acc.at[pl.ds(base + 1024, 1024)])
    pltpu.sync_copy(zeros.at[:], acc.at[pl.ds(base + 2048, 1024)])
    pltpu.sync_copy(zeros.at[pl.ds(0, 64)], acc.at[pl.ds(base + 3072, 64)])
    plsc.subcore_barrier()

    tpb_half = TPB // 2
    npair = tpb_half // (2 * K)

    def _load(chunk, slot):
        blk0 = c * (EB // 2) + t * tpb_half + chunk * K
        pltpu.sync_copy(colb.at[pl.ds(blk0, K)], cbuf.at[pl.ds(slot * K, K)])

    def _scat(slot, sem):
        for j in range(K):
            pltpu.async_copy(ones, acc.at[cbuf.at[slot * K + j]], sem,
                             add=True)

    def _wait(sem):
        for _ in range(K):
            pltpu.make_async_copy(
                deg_out.at[pl.ds(0, 128)], zeros.at[pl.ds(0, 128)], sem
            ).wait()

    _load(0, 0)

    def _pair(ip, _):
        a2 = ip * 2
        _scat(0, ds0)

        @pl.when(ip > 0)
        def _():
            _wait(ds1)

        _load(a2 + 1, 1)
        _scat(1, ds1)

        @pl.when(ip < npair - 1)
        def _():
            _wait(ds0)
            _load(a2 + 2, 0)

        return 0

    lax.fori_loop(0, npair, _pair, 0)
    _wait(ds0)
    _wait(ds1)
    plsc.subcore_barrier()
    for off, sz in ((0, 1024), (1024, 1024), (2048, 1024), (3072, 64)):
        pltpu.sync_copy(acc.at[pl.ds(base + off, sz)], zeros.at[pl.ds(0, sz)])
        pltpu.sync_copy(
            zeros.at[pl.ds(0, sz)], deg_out.at[pl.ds(c * NP + base + off, sz)]
        )


@functools.partial(
    pl.kernel,
    out_type=[
        jax.ShapeDtypeStruct((2 * NP, HALF), _F32),
        jax.ShapeDtypeStruct((2 * NP, HALF), _F32),
        jax.ShapeDtypeStruct((2 * NP, HALF), _F32),
        jax.ShapeDtypeStruct((2 * NP, HALF), _F32),
    ],
    mesh=_sc_mesh(),
    compiler_params=pltpu.CompilerParams(use_tc_tiling_on_sc=False),
    scratch_types=[
        pltpu.VMEM((CHK,), jnp.int32),
        pltpu.VMEM((CHK,), jnp.int32),
        pltpu.VMEM((CHK,), jnp.int32),
        pltpu.VMEM((CHK,), jnp.int32),
        pltpu.VMEM((CHK,), jnp.int32),
        pltpu.VMEM((CHK,), jnp.int32),
        pltpu.VMEM((CHK,), jnp.int32),
        pltpu.VMEM((CHK,), jnp.int32),
        pltpu.VMEM((CHK, HALF), _F32),
        pltpu.VMEM((CHK, HALF), _F32),
        pltpu.VMEM((ZR + 16,), _F32),
        pltpu.VMEM_SHARED((NP, HALF), _F32),
        pltpu.SemaphoreType.DMA,
        pltpu.SemaphoreType.DMA,
        pltpu.SemaphoreType.DMA,
        pltpu.SemaphoreType.DMA,
        pltpu.SemaphoreType.DMA,
        pltpu.SemaphoreType.DMA,
        pltpu.SemaphoreType.DMA,
        pltpu.SemaphoreType.DMA,
    ],
)
def _prop3_kernel(emb2, s, colf, rowf, e1, e2, e3, fb, cb0, cb1, cb2, cb3,
                  rb0, rb1, rb2, rb3, vbufA, vbufB, sv, acc,
                  is0, is1, is2, is3, gs0, gs1, ss0, ss1):
    c = lax.axis_index("c")
    t = lax.axis_index("s")
    coff = c * NP
    base = t * ZR

    pltpu.sync_copy(s.at[pl.ds(base, ZR)], sv.at[pl.ds(0, ZR)])

    def _scale_rows(n_rows):
        def _sr(r, off_r):
            sc = jnp.full((16,), sv[pl.ds(off_r + r, 16)][0], _F32)
            vbufA[r, pl.ds(0, 16)] = vbufA[r, pl.ds(0, 16)] * sc
            vbufA[r, pl.ds(16, 16)] = vbufA[r, pl.ds(16, 16)] * sc
            return off_r

        return _sr

    for off, sz in _ZCHUNKS:
        pltpu.sync_copy(emb2.at[pl.ds(coff + base + off, sz)],
                        vbufA.at[pl.ds(0, sz)])
        lax.fori_loop(0, sz, _scale_rows(sz), off)
        pltpu.sync_copy(vbufA.at[pl.ds(0, sz)],
                        fb.at[pl.ds(coff + base + off, sz)])
    plsc.subcore_barrier()

    ebase = t * ETILE
    cbufs = (cb0, cb1, cb2, cb3)
    rbufs = (rb0, rb1, rb2, rb3)
    isems = (is0, is1, is2, is3)
    vbufs = (vbufA, vbufB)
    gsems = (gs0, gs1)
    ssems = (ss0, ss1)

    def _load(chunk, q):
        eoff = ebase + chunk * CHK
        pltpu.async_copy(colf.at[pl.ds(c * EPAD + eoff, CHK)], cbufs[q],
                         isems[q])
        pltpu.async_copy(rowf.at[pl.ds(eoff, CHK)], rbufs[q], isems[q])

    def _wait_idx(q):
        for _ in range(2):
            pltpu.make_async_copy(
                colf.at[pl.ds(0, CHK)], cbufs[q], isems[q]
            ).wait()

    def _wait1(sem):
        pltpu.make_async_copy(fb.at[pl.ds(0, CHK)], vbufA, sem).wait()

    for layer, e_out in enumerate((e1, e2, e3)):
        def _zb(i, _):
            vbufA[i, pl.ds(0, 16)] = jnp.zeros((16,), _F32)
            vbufA[i, pl.ds(16, 16)] = jnp.zeros((16,), _F32)
            return 0

        lax.fori_loop(0, CHK, _zb, 0)
        for off, sz in _ZCHUNKS:
            pltpu.sync_copy(vbufA.at[pl.ds(0, sz)],
                            acc.at[pl.ds(base + off, sz)])
        plsc.subcore_barrier()

        _load(0, 0)
        _load(1, 1)

        def _step(i0, g, k, skip_prefetch, static_mid):
            p = k % 2
            o = 1 - p

            def _s1():
                _wait1(ssems[p])

            def _s5():
                _wait1(gsems[o])
                pltpu.async_copy(vbufs[o], acc.at[rbufs[(k - 1) % 4]],
                                 ssems[o], add=True)

            if k >= 2 or static_mid:
                _s1()
            else:
                pl.when(g > 0)(_s1)
            if not (skip_prefetch and k >= 2):
                _load(i0 + k + 2, (k + 2) % 4)
            _wait_idx(k)
            pltpu.async_copy(fb.at[cbufs[k]], vbufs[p], gsems[p])
            if k >= 1 or static_mid:
                _s5()
            else:
                pl.when(g > 0)(_s5)

        ngrp = NCH // 4

        def _grp(g, _):
            for k in range(4):
                _step(g * 4, g, k, False, False)
            return 0

        lax.fori_loop(0, ngrp - 1, _grp, 0)
        for k in range(4):
            _step((ngrp - 1) * 4, ngrp - 1, k, True, True)
        _wait1(gsems[1])
        pltpu.async_copy(vbufs[1], acc.at[rbufs[3]], ssems[1], add=True)
        _wait1(ssems[0])
        _wait1(ssems[1])
        plsc.subcore_barrier()

        for off, sz in _ZCHUNKS:
            pltpu.sync_copy(acc.at[pl.ds(base + off, sz)],
                            vbufA.at[pl.ds(0, sz)])
            lax.fori_loop(0, sz, _scale_rows(sz), off)
            pltpu.sync_copy(vbufA.at[pl.ds(0, sz)],
                            e_out.at[pl.ds(coff + base + off, sz)])
            if layer < 2:
                lax.fori_loop(0, sz, _scale_rows(sz), off)
                pltpu.sync_copy(vbufA.at[pl.ds(0, sz)],
                                fb.at[pl.ds(coff + base + off, sz)])
        if layer < 2:
            plsc.subcore_barrier()


def _mlp_body(x_ref, e_ref, wft_ref, w1_ref, w2_ref, bf_ref, bc_ref, o_ref):
    p = jnp.maximum(
        jnp.dot(x_ref[...], wft_ref[...], preferred_element_type=_F32)
        + bf_ref[...],
        0.0,
    )
    o = jnp.maximum(
        jnp.dot(e_ref[...], w1_ref[...], preferred_element_type=_F32)
        + jnp.dot(p, w2_ref[...], preferred_element_type=_F32)
        + bc_ref[...],
        0.0,
    )
    o_ref[...] = o


def _mlp(x, emb, wft, w1, w2, bf, bc):
    n = x.shape[0]
    grid = (n + 127) // 128
    return pl.pallas_call(
        _mlp_body,
        grid=(grid,),
        in_specs=[
            pl.BlockSpec((128, UFD), lambda b: (b, 0)),
            pl.BlockSpec((128, EMB), lambda b: (b, 0)),
            pl.BlockSpec((UFD, EMB), lambda b: (0, 0)),
            pl.BlockSpec((EMB, EMB), lambda b: (0, 0)),
            pl.BlockSpec((EMB, EMB), lambda b: (0, 0)),
            pl.BlockSpec((1, EMB), lambda b: (0, 0)),
            pl.BlockSpec((1, EMB), lambda b: (0, 0)),
        ],
        out_specs=pl.BlockSpec((128, EMB), lambda b: (b, 0)),
        out_shape=jax.ShapeDtypeStruct((n, EMB), _F32),
    )(x, emb, wft, w1, w2, bf, bc)


def _s_body(d0_ref, d1_ref, s_ref):
    b = pl.program_id(0)
    deg = d0_ref[...] + d1_ref[...]
    rows = b * 512 + lax.broadcasted_iota(jnp.int32, (512,), 0)
    valid = (rows < NN) & (deg > 0.0)
    s_ref[...] = jnp.where(valid, lax.rsqrt(jnp.maximum(deg, 1e-30)), 0.0)


def _s_kernel(degp):
    nb = NP // 512
    return pl.pallas_call(
        _s_body,
        grid=(nb,),
        in_specs=[
            pl.BlockSpec((512,), lambda b: (b,)),
            pl.BlockSpec((512,), lambda b, _nb=nb: (_nb + b,)),
        ],
        out_specs=pl.BlockSpec((512,), lambda b: (b,)),
        out_shape=jax.ShapeDtypeStruct((NP,), _F32),
    )(degp, degp)


def _final_body(a_ref, b_ref, c_ref, d_ref, o_ref):
    o_ref[...] = 0.25 * (a_ref[...] + b_ref[...] + c_ref[...] + d_ref[...])


def _final(emb2, e1, e2, e3):
    rows = 2 * NP * HALF // 128
    view = lambda x: x.reshape(rows, 128)
    spec = pl.BlockSpec((512, 128), lambda b: (b, 0))
    out = pl.pallas_call(
        _final_body,
        grid=(rows // 512,),
        in_specs=[spec, spec, spec, spec],
        out_specs=spec,
        out_shape=jax.ShapeDtypeStruct((rows, 128), _F32),
    )(view(emb2), view(e1), view(e2), view(e3))
    return out.reshape(2 * NP, HALF)


@jax.jit
def kernel(edge_index, user_features, item_features, user_emb_table,
           item_emb_table, Wuf, buf, Wuc, buc, Wif, bif, Wic, bic):
    row = edge_index[0]
    col = edge_index[1]
    pad = jnp.full((EPAD - EE,), NN, jnp.int32)
    colb = jnp.concatenate([col, pad]).reshape(EB, 128)

    u_emb = _mlp(user_features, user_emb_table, Wuf.T,
                 Wuc[:, :EMB].T, Wuc[:, EMB:].T,
                 buf.reshape(1, EMB), buc.reshape(1, EMB))
    i_emb = _mlp(item_features, item_emb_table, Wif.T,
                 Wic[:, :EMB].T, Wic[:, EMB:].T,
                 bif.reshape(1, EMB), bic.reshape(1, EMB))

    degp = _deg_kernel(colb)
    s = _s_kernel(degp)

    zpad = jnp.zeros((NP - NN, HALF), _F32)
    emb2 = jnp.concatenate(
        [u_emb[:, :HALF], i_emb[:, :HALF], zpad,
         u_emb[:, HALF:], i_emb[:, HALF:], zpad], axis=0
    )

    colp = jnp.concatenate([col, pad])
    rowp = jnp.concatenate([row, pad])
    colf = jnp.concatenate([colp, colp + NP])
    e1, e2, e3, _ = _prop3_kernel(emb2, s, colf, rowp)

    fin2 = _final(emb2, e1, e2, e3)
    users = jnp.concatenate([fin2[:NUSR], fin2[NP:NP + NUSR]], axis=1)
    items = jnp.concatenate(
        [fin2[NUSR:NN], fin2[NP + NUSR:NP + NN]], axis=1
    )
    return users, items
